# baseline (speedup 1.0000x reference)
import functools
import math

import jax
import jax.numpy as jnp
from jax import lax
from jax.experimental import pallas as pl
from jax.experimental.pallas import tpu as pltpu
from jax.experimental.pallas import tpu_sc as plsc


def _router_body(x_ref, gw_ref, p_ref):
    g = lax.dot_general(
        gw_ref[...], x_ref[...], (((1,), (1,)), ((), ())),
        preferred_element_type=jnp.float32,
    )
    m = jnp.max(g, axis=0, keepdims=True)
    ex = jnp.exp(g - m)
    p_ref[...] = ex / jnp.sum(ex, axis=0, keepdims=True)


def _router(x, gate_w, bm=1024):
    n, h = x.shape
    e = gate_w.shape[0]
    return pl.pallas_call(
        _router_body,
        grid=(n // bm,),
        in_specs=[
            pl.BlockSpec((bm, h), lambda m: (m, 0)),
            pl.BlockSpec((e, h), lambda m: (0, 0)),
        ],
        out_specs=pl.BlockSpec((e, bm), lambda m: (0, m)),
        out_shape=jax.ShapeDtypeStruct((e, n), jnp.float32),
    )(x, gate_w)


def _threshold_body(k, p_ref, thr_ref, cgt_ref):
    bits = lax.bitcast_convert_type(p_ref[...], jnp.int32)
    e = bits.shape[0]
    t = jnp.zeros((e, 1), jnp.int32)
    for b in range(30, -1, -1):
        cand = t | jnp.int32(1 << b)
        cnt = jnp.sum((bits >= cand).astype(jnp.int32), axis=1, keepdims=True)
        t = jnp.where(cnt >= k, cand, t)
    cgt = jnp.sum((bits > t).astype(jnp.int32), axis=1, keepdims=True)
    thr_ref[...] = jnp.broadcast_to(lax.bitcast_convert_type(t, jnp.float32), (e, 16))
    cgt_ref[...] = jnp.broadcast_to(cgt, (e, 16))


def _thresholds(probs_t, k):
    e, n = probs_t.shape
    return pl.pallas_call(
        functools.partial(_threshold_body, k),
        in_specs=[pl.BlockSpec((e, n), lambda: (0, 0))],
        out_specs=[
            pl.BlockSpec((e, 16), lambda: (0, 0)),
            pl.BlockSpec((e, 16), lambda: (0, 0)),
        ],
        out_shape=[
            jax.ShapeDtypeStruct((e, 16), jnp.float32),
            jax.ShapeDtypeStruct((e, 16), jnp.int32),
        ],
    )(probs_t)


def _compact_body(e_num, n, k, p_hbm, thr_hbm, cgt_hbm, idx_hbm, sc_hbm,
                  pv, iv, sv, tv, cv):
    cid = lax.axis_index("c")
    sid = lax.axis_index("s")
    wid = sid * 2 + cid

    @pl.when(wid < e_num)
    def _():
        pltpu.sync_copy(p_hbm.at[wid], pv)
        pltpu.sync_copy(thr_hbm.at[wid], tv)
        pltpu.sync_copy(cgt_hbm.at[wid], cv)
        t = tv[...]
        cgt = cv[...]
        r = k - cgt

        def body(i, carry):
            cg, ce = carry
            v = pv[pl.ds(i * 16, 16)]
            gt = v > t
            eq = v == t
            pgt = plsc.cumsum(gt.astype(jnp.int32)) + cg
            peq = plsc.cumsum(eq.astype(jnp.int32)) + ce
            sel_eq = eq & (peq <= r)
            slot = jnp.where(gt, pgt - 1, cgt + peq - 1)
            m = gt | sel_eq
            ids = lax.iota(jnp.int32, 16) + i * 16
            plsc.store_scatter(iv, [slot], ids, mask=m)
            plsc.store_scatter(sv, [slot], v, mask=m)
            return (cg + plsc.all_reduce_population_count(gt),
                    ce + plsc.all_reduce_population_count(eq))

        zero16 = jnp.zeros((16,), jnp.int32)
        lax.fori_loop(0, n // 16, body, (zero16, zero16))
        pltpu.sync_copy(iv, idx_hbm.at[wid])
        pltpu.sync_copy(sv, sc_hbm.at[wid])


def _compact(probs_t, thr, cgt, k):
    e, n = probs_t.shape
    mesh = plsc.VectorSubcoreMesh(core_axis_name="c", subcore_axis_name="s")
    f = functools.partial(
        pl.kernel,
        out_type=(jax.ShapeDtypeStruct((e, k), jnp.int32),
                  jax.ShapeDtypeStruct((e, k), jnp.float32)),
        mesh=mesh,
        scratch_types=[
            pltpu.VMEM((n,), jnp.float32),
            pltpu.VMEM((k,), jnp.int32),
            pltpu.VMEM((k,), jnp.float32),
            pltpu.VMEM((16,), jnp.float32),
            pltpu.VMEM((16,), jnp.int32),
        ],
        compiler_params=pltpu.CompilerParams(needs_layout_passes=False),
    )(functools.partial(_compact_body, e, n, k))
    return f(probs_t, thr, cgt)


def _gather_body(rows_w, ch, x_hbm, idx_hbm, xg_hbm, idxv, buf, sem):
    cid = lax.axis_index("c")
    sid = lax.axis_index("s")
    wid = sid * 2 + cid
    base = wid * rows_w
    pltpu.sync_copy(idx_hbm.at[pl.ds(base, rows_w)], idxv)

    def chunk(c, _):
        pltpu.async_copy(x_hbm.at[idxv.at[pl.ds(c * ch, ch)]], buf, sem).wait()
        pltpu.sync_copy(buf, xg_hbm.at[pl.ds(base + c * ch, ch)])
        return 0

    lax.fori_loop(0, rows_w // ch, chunk, 0)


def _gather(x, idx_flat, ch=32):
    n, h = x.shape
    total = idx_flat.shape[0]
    rows_w = total // 32
    mesh = plsc.VectorSubcoreMesh(core_axis_name="c", subcore_axis_name="s")
    f = functools.partial(
        pl.kernel,
        out_type=jax.ShapeDtypeStruct((total, h), jnp.float32),
        mesh=mesh,
        scratch_types=[
            pltpu.VMEM((rows_w,), jnp.int32),
            pltpu.VMEM((ch, h), jnp.float32),
            pltpu.SemaphoreType.DMA,
        ],
        compiler_params=pltpu.CompilerParams(needs_layout_passes=False),
    )(functools.partial(_gather_body, rows_w, ch))
    return f(x, idx_flat)


def _mlp_a_body(xg_ref, w1_ref, b1_ref, h_ref, xb_ref):
    f_idx = pl.program_id(1)

    @pl.when(f_idx == 0)
    def _():
        xb_ref[...] = xg_ref[0].astype(jnp.bfloat16)

    w1b = w1_ref[0].astype(jnp.bfloat16)
    h = lax.dot_general(xb_ref[...], w1b, (((1,), (1,)), ((), ())),
                        preferred_element_type=jnp.float32)
    h = h + b1_ref[0, 0, :][None, :]
    h = 0.5 * h * (1.0 + lax.erf(h * (1.0 / math.sqrt(2.0))))
    h_ref[0] = h.astype(jnp.bfloat16)


def _mlp_a(xg, w1, b1, bf=512):
    e, k, h = xg.shape
    f = w1.shape[1]
    return pl.pallas_call(
        _mlp_a_body,
        grid=(e, f // bf),
        in_specs=[
            pl.BlockSpec((1, k, h), lambda ei, fi: (ei, 0, 0)),
            pl.BlockSpec((1, bf, h), lambda ei, fi: (ei, fi, 0)),
            pl.BlockSpec((1, 1, bf), lambda ei, fi: (ei, 0, fi)),
        ],
        out_specs=pl.BlockSpec((1, k, bf), lambda ei, fi: (ei, 0, fi)),
        out_shape=jax.ShapeDtypeStruct((e, k, f), jnp.bfloat16),
        scratch_shapes=[pltpu.VMEM((k, h), jnp.bfloat16)],
    )(xg, w1, b1)


def _mlp_b_body(fsteps, h_ref, w2_ref, b2_ref, s_ref, yg_ref):
    f_idx = pl.program_id(2)
    w2b = w2_ref[0].astype(jnp.bfloat16)
    contrib = lax.dot_general(h_ref[0], w2b, (((1,), (1,)), ((), ())),
                              preferred_element_type=jnp.float32)

    @pl.when(f_idx == 0)
    def _():
        yg_ref[0] = contrib

    @pl.when(f_idx > 0)
    def _():
        yg_ref[0] += contrib

    @pl.when(f_idx == fsteps - 1)
    def _():
        yg_ref[0] = (yg_ref[0] + b2_ref[0, 0, :][None, :]) * s_ref[0]


def _mlp_b(hh, w2, b2, scores3, bf=256, nh=2048):
    e, k, f = hh.shape
    h = w2.shape[1]
    fsteps = f // bf
    return pl.pallas_call(
        functools.partial(_mlp_b_body, fsteps),
        grid=(e, h // nh, fsteps),
        in_specs=[
            pl.BlockSpec((1, k, bf), lambda ei, ni, fi: (ei, 0, fi)),
            pl.BlockSpec((1, nh, bf), lambda ei, ni, fi: (ei, ni, fi)),
            pl.BlockSpec((1, 1, nh), lambda ei, ni, fi: (ei, 0, ni)),
            pl.BlockSpec((1, k, 1), lambda ei, ni, fi: (ei, 0, 0)),
        ],
        out_specs=pl.BlockSpec((1, k, nh), lambda ei, ni, fi: (ei, 0, ni)),
        out_shape=jax.ShapeDtypeStruct((e, k, h), jnp.float32),
    )(hh, w2, b2, scores3)


def _scatter_body(bt, idx_ref, yg_ref, out_ref):
    t, e_idx, s_idx = pl.program_id(0), pl.program_id(1), pl.program_id(2)
    bs = idx_ref.shape[2]
    tok = lax.broadcasted_iota(jnp.int32, (bt, bs), 0) + t * bt
    p = (tok == idx_ref[0, 0, :][None, :]).astype(jnp.bfloat16)
    ygb = yg_ref[0].astype(jnp.bfloat16)
    contrib = lax.dot_general(p, ygb, (((1,), (0,)), ((), ())),
                              preferred_element_type=jnp.float32)
    first = (e_idx == 0) & (s_idx == 0)

    @pl.when(first)
    def _():
        out_ref[...] = contrib

    @pl.when(jnp.logical_not(first))
    def _():
        out_ref[...] += contrib


def _scatter(idx3, yg, n, bt=1024, bs=256):
    e, k, h = yg.shape
    return pl.pallas_call(
        functools.partial(_scatter_body, bt),
        grid=(n // bt, e, k // bs),
        in_specs=[
            pl.BlockSpec((1, 1, bs), lambda t, ei, si: (ei, 0, si)),
            pl.BlockSpec((1, bs, h), lambda t, ei, si: (ei, si, 0)),
        ],
        out_specs=pl.BlockSpec((bt, h), lambda t, ei, si: (t, 0)),
        out_shape=jax.ShapeDtypeStruct((n, h), jnp.float32),
    )(idx3, yg)


def _scat_body(total, tw, sb_num, yg_hbm, idx_hbm, z_hbm, out_hbm,
               idxv, rowl, tokl, buf, acc, sem):
    cid = lax.axis_index("c")
    sid = lax.axis_index("s")
    wid = sid * 2 + cid
    wbase = wid * (tw * sb_num)
    pltpu.sync_copy(idx_hbm, idxv)
    lanes = lax.iota(jnp.int32, 16)

    def sub_block(sb, _):
        lo = wbase + sb * tw
        pltpu.sync_copy(z_hbm, acc)

        def fill(i, _):
            rowl[pl.ds(i * 16, 16)] = i * 16 + lanes
            return 0
        lax.fori_loop(0, rowl.shape[0] // 16, fill, 0)

        def scan(i, cnt):
            v = idxv[pl.ds(i * 16, 16)]
            m = (v >= lo) & (v < lo + tw)
            pos = plsc.cumsum(m.astype(jnp.int32)) + cnt - 1
            rows = i * 16 + lanes
            plsc.store_scatter(tokl, [pos], v - lo, mask=m)
            plsc.store_scatter(rowl, [pos], rows, mask=m)
            return cnt + plsc.all_reduce_population_count(m)
        cnt = lax.fori_loop(0, total // 16, scan, jnp.zeros((16,), jnp.int32))
        nw = lax.reduce_max(cnt, (0,))

        def g(gi, _):
            rows_v = rowl[pl.ds(gi * 16, 16)]
            pltpu.async_copy(yg_hbm.at[rows_v], buf, sem).wait()
            ta = tokl[pl.ds(gi * 16, 16)]

            def add_row(j, _):
                tok = lax.reduce_max(jnp.where(lanes == j, ta, -1), (0,))
                for c in range(buf.shape[1] // 16):
                    plsc.addupdate(acc.at[tok, pl.ds(c * 16, 16)],
                                   buf[j, pl.ds(c * 16, 16)])
                return 0
            lax.fori_loop(0, jnp.minimum(nw - gi * 16, 16), add_row, 0)
            return 0
        lax.fori_loop(0, (nw + 15) // 16, g, 0)
        pltpu.sync_copy(acc, out_hbm.at[pl.ds(lo, tw)])
        return 0

    lax.fori_loop(0, sb_num, sub_block, 0)


def _scatter_sc(yg_flat, idx_flat, n, tw=32):
    total, h = yg_flat.shape
    zeros = jnp.zeros((tw, h), jnp.float32)
    sb_num = n // (tw * 32)
    lcap = 8 * tw
    mesh = plsc.VectorSubcoreMesh(core_axis_name="c", subcore_axis_name="s")
    f = functools.partial(
        pl.kernel,
        out_type=jax.ShapeDtypeStruct((n, h), jnp.float32),
        mesh=mesh,
        scratch_types=[
            pltpu.VMEM((total,), jnp.int32),
            pltpu.VMEM((lcap,), jnp.int32),
            pltpu.VMEM((lcap,), jnp.int32),
            pltpu.VMEM((16, h), jnp.float32),
            pltpu.VMEM((tw, h), jnp.float32),
            pltpu.SemaphoreType.DMA,
        ],
        compiler_params=pltpu.CompilerParams(needs_layout_passes=False),
    )(functools.partial(_scat_body, total, tw, sb_num))
    return f(yg_flat, idx_flat, zeros)


def _loss_body(total, s_ref, o_ref):
    sums = jnp.sum(s_ref[...], axis=1, keepdims=True)
    loads = sums / total
    e = loads.shape[0]
    m = jnp.sum(loads, axis=0, keepdims=True) / e
    d = loads - m
    o_ref[...] = jnp.sum(d * d, axis=0, keepdims=True) * (e / (e - 1.0))


def _loss(scores, total):
    e, k = scores.shape
    return pl.pallas_call(
        functools.partial(_loss_body, float(total)),
        in_specs=[pl.BlockSpec((e, k), lambda: (0, 0))],
        out_specs=pl.BlockSpec((1, 1), lambda: (0, 0)),
        out_shape=jax.ShapeDtypeStruct((1, 1), jnp.float32),
    )(scores)


def kernel(hidden_states, gate_w, w1, b1, w2, b2):
    b, s, h = hidden_states.shape
    e = gate_w.shape[0]
    n = b * s
    k = int(1.25 * (n // e))

    x = hidden_states.reshape(n, h)
    probs_t = _router(x, gate_w)
    thr, cgt = _thresholds(probs_t, k)
    idx, scores = _compact(probs_t, thr, cgt, k)
    xg = _gather(x, idx.reshape(e * k))
    hh = _mlp_a(xg.reshape(e, k, h), w1, b1.reshape(e, 1, -1))
    yg = _mlp_b(hh, w2, b2.reshape(e, 1, -1), scores.reshape(e, k, 1))
    out = _scatter_sc(yg.reshape(e * k, h), idx.reshape(e * k), n)
    loss = _loss(scores, n)
    return (out.reshape(b, s, h), loss.reshape(()))


# speedup vs baseline: 1.1144x; 1.1144x over previous
import functools
import math

import jax
import jax.numpy as jnp
from jax import lax
from jax.experimental import pallas as pl
from jax.experimental.pallas import tpu as pltpu
from jax.experimental.pallas import tpu_sc as plsc


def _router_body(x_ref, gw_ref, p_ref):
    g = lax.dot_general(
        gw_ref[...], x_ref[...], (((1,), (1,)), ((), ())),
        preferred_element_type=jnp.float32,
    )
    m = jnp.max(g, axis=0, keepdims=True)
    ex = jnp.exp(g - m)
    p_ref[...] = ex / jnp.sum(ex, axis=0, keepdims=True)


def _router(x, gate_w, bm=1024):
    n, h = x.shape
    e = gate_w.shape[0]
    return pl.pallas_call(
        _router_body,
        grid=(n // bm,),
        in_specs=[
            pl.BlockSpec((bm, h), lambda m: (m, 0)),
            pl.BlockSpec((e, h), lambda m: (0, 0)),
        ],
        out_specs=pl.BlockSpec((e, bm), lambda m: (0, m)),
        out_shape=jax.ShapeDtypeStruct((e, n), jnp.float32),
    )(x, gate_w)


def _threshold_body(k, p_ref, thr_ref, cgt_ref):
    bits = lax.bitcast_convert_type(p_ref[...], jnp.int32)
    e = bits.shape[0]
    t = jnp.zeros((e, 1), jnp.int32)
    for b in range(30, -1, -1):
        cand = t | jnp.int32(1 << b)
        cnt = jnp.sum((bits >= cand).astype(jnp.int32), axis=1, keepdims=True)
        t = jnp.where(cnt >= k, cand, t)
    cgt = jnp.sum((bits > t).astype(jnp.int32), axis=1, keepdims=True)
    thr_ref[...] = jnp.broadcast_to(lax.bitcast_convert_type(t, jnp.float32), (e, 16))
    cgt_ref[...] = jnp.broadcast_to(cgt, (e, 16))


def _thresholds(probs_t, k):
    e, n = probs_t.shape
    return pl.pallas_call(
        functools.partial(_threshold_body, k),
        in_specs=[pl.BlockSpec((e, n), lambda: (0, 0))],
        out_specs=[
            pl.BlockSpec((e, 16), lambda: (0, 0)),
            pl.BlockSpec((e, 16), lambda: (0, 0)),
        ],
        out_shape=[
            jax.ShapeDtypeStruct((e, 16), jnp.float32),
            jax.ShapeDtypeStruct((e, 16), jnp.int32),
        ],
    )(probs_t)


def _compact_body(e_num, n, k, p_hbm, thr_hbm, cgt_hbm, idx_hbm, sc_hbm,
                  pv, iv, sv, tv, cv):
    cid = lax.axis_index("c")
    sid = lax.axis_index("s")
    wid = sid * 2 + cid

    @pl.when(wid < e_num)
    def _():
        pltpu.sync_copy(p_hbm.at[wid], pv)
        pltpu.sync_copy(thr_hbm.at[wid], tv)
        pltpu.sync_copy(cgt_hbm.at[wid], cv)
        t = tv[...]
        cgt = cv[...]
        r = k - cgt

        def body(i, carry):
            cg, ce = carry
            v = pv[pl.ds(i * 16, 16)]
            gt = v > t
            eq = v == t
            pgt = plsc.cumsum(gt.astype(jnp.int32)) + cg
            peq = plsc.cumsum(eq.astype(jnp.int32)) + ce
            sel_eq = eq & (peq <= r)
            slot = jnp.where(gt, pgt - 1, cgt + peq - 1)
            m = gt | sel_eq
            ids = lax.iota(jnp.int32, 16) + i * 16
            plsc.store_scatter(iv, [slot], ids, mask=m)
            plsc.store_scatter(sv, [slot], v, mask=m)
            return (cg + plsc.all_reduce_population_count(gt),
                    ce + plsc.all_reduce_population_count(eq))

        zero16 = jnp.zeros((16,), jnp.int32)
        lax.fori_loop(0, n // 16, body, (zero16, zero16))
        pltpu.sync_copy(iv, idx_hbm.at[wid])
        pltpu.sync_copy(sv, sc_hbm.at[wid])


def _compact(probs_t, thr, cgt, k):
    e, n = probs_t.shape
    mesh = plsc.VectorSubcoreMesh(core_axis_name="c", subcore_axis_name="s")
    f = functools.partial(
        pl.kernel,
        out_type=(jax.ShapeDtypeStruct((e, k), jnp.int32),
                  jax.ShapeDtypeStruct((e, k), jnp.float32)),
        mesh=mesh,
        scratch_types=[
            pltpu.VMEM((n,), jnp.float32),
            pltpu.VMEM((k,), jnp.int32),
            pltpu.VMEM((k,), jnp.float32),
            pltpu.VMEM((16,), jnp.float32),
            pltpu.VMEM((16,), jnp.int32),
        ],
        compiler_params=pltpu.CompilerParams(needs_layout_passes=False),
    )(functools.partial(_compact_body, e, n, k))
    return f(probs_t, thr, cgt)


def _gather_body(rows_w, ch, x_hbm, idx_hbm, xg_hbm, idxv, buf, sem):
    cid = lax.axis_index("c")
    sid = lax.axis_index("s")
    wid = sid * 2 + cid
    base = wid * rows_w
    pltpu.sync_copy(idx_hbm.at[pl.ds(base, rows_w)], idxv)

    def chunk(c, _):
        pltpu.async_copy(x_hbm.at[idxv.at[pl.ds(c * ch, ch)]], buf, sem).wait()
        pltpu.sync_copy(buf, xg_hbm.at[pl.ds(base + c * ch, ch)])
        return 0

    lax.fori_loop(0, rows_w // ch, chunk, 0)


def _gather(x, idx_flat, ch=32):
    n, h = x.shape
    total = idx_flat.shape[0]
    rows_w = total // 32
    mesh = plsc.VectorSubcoreMesh(core_axis_name="c", subcore_axis_name="s")
    f = functools.partial(
        pl.kernel,
        out_type=jax.ShapeDtypeStruct((total, h), jnp.float32),
        mesh=mesh,
        scratch_types=[
            pltpu.VMEM((rows_w,), jnp.int32),
            pltpu.VMEM((ch, h), jnp.float32),
            pltpu.SemaphoreType.DMA,
        ],
        compiler_params=pltpu.CompilerParams(needs_layout_passes=False),
    )(functools.partial(_gather_body, rows_w, ch))
    return f(x, idx_flat)


def _mlp_a_body(xg_ref, w1_ref, b1_ref, h_ref, xb_ref):
    f_idx = pl.program_id(1)

    @pl.when(f_idx == 0)
    def _():
        xb_ref[...] = xg_ref[0].astype(jnp.bfloat16)

    w1b = w1_ref[0].astype(jnp.bfloat16)
    h = lax.dot_general(xb_ref[...], w1b, (((1,), (1,)), ((), ())),
                        preferred_element_type=jnp.float32)
    h = h + b1_ref[0, 0, :][None, :]
    h = 0.5 * h * (1.0 + lax.erf(h * (1.0 / math.sqrt(2.0))))
    h_ref[0] = h.astype(jnp.bfloat16)


def _mlp_a(xg, w1, b1, bf=512):
    e, k, h = xg.shape
    f = w1.shape[1]
    return pl.pallas_call(
        _mlp_a_body,
        grid=(e, f // bf),
        in_specs=[
            pl.BlockSpec((1, k, h), lambda ei, fi: (ei, 0, 0)),
            pl.BlockSpec((1, bf, h), lambda ei, fi: (ei, fi, 0)),
            pl.BlockSpec((1, 1, bf), lambda ei, fi: (ei, 0, fi)),
        ],
        out_specs=pl.BlockSpec((1, k, bf), lambda ei, fi: (ei, 0, fi)),
        out_shape=jax.ShapeDtypeStruct((e, k, f), jnp.bfloat16),
        scratch_shapes=[pltpu.VMEM((k, h), jnp.bfloat16)],
    )(xg, w1, b1)


def _mlp_b_body(fsteps, h_ref, w2_ref, b2_ref, s_ref, yg_ref):
    f_idx = pl.program_id(2)
    w2b = w2_ref[0].astype(jnp.bfloat16)
    contrib = lax.dot_general(h_ref[0], w2b, (((1,), (1,)), ((), ())),
                              preferred_element_type=jnp.float32)

    @pl.when(f_idx == 0)
    def _():
        yg_ref[0] = contrib

    @pl.when(f_idx > 0)
    def _():
        yg_ref[0] += contrib

    @pl.when(f_idx == fsteps - 1)
    def _():
        yg_ref[0] = (yg_ref[0] + b2_ref[0, 0, :][None, :]) * s_ref[0]


def _mlp_b(hh, w2, b2, scores3, bf=512, nh=2048):
    e, k, f = hh.shape
    h = w2.shape[1]
    fsteps = f // bf
    return pl.pallas_call(
        functools.partial(_mlp_b_body, fsteps),
        grid=(e, h // nh, fsteps),
        in_specs=[
            pl.BlockSpec((1, k, bf), lambda ei, ni, fi: (ei, 0, fi)),
            pl.BlockSpec((1, nh, bf), lambda ei, ni, fi: (ei, ni, fi)),
            pl.BlockSpec((1, 1, nh), lambda ei, ni, fi: (ei, 0, ni)),
            pl.BlockSpec((1, k, 1), lambda ei, ni, fi: (ei, 0, 0)),
        ],
        out_specs=pl.BlockSpec((1, k, nh), lambda ei, ni, fi: (ei, 0, ni)),
        out_shape=jax.ShapeDtypeStruct((e, k, h), jnp.float32),
    )(hh, w2, b2, scores3)


def _scatter_body(bt, idx_ref, yg_ref, out_ref):
    t, e_idx, s_idx = pl.program_id(0), pl.program_id(1), pl.program_id(2)
    bs = idx_ref.shape[2]
    tok = lax.broadcasted_iota(jnp.int32, (bt, bs), 0) + t * bt
    p = (tok == idx_ref[0, 0, :][None, :]).astype(jnp.bfloat16)
    ygb = yg_ref[0].astype(jnp.bfloat16)
    contrib = lax.dot_general(p, ygb, (((1,), (0,)), ((), ())),
                              preferred_element_type=jnp.float32)
    first = (e_idx == 0) & (s_idx == 0)

    @pl.when(first)
    def _():
        out_ref[...] = contrib

    @pl.when(jnp.logical_not(first))
    def _():
        out_ref[...] += contrib


def _scatter(idx3, yg, n, bt=1024, bs=256):
    e, k, h = yg.shape
    return pl.pallas_call(
        functools.partial(_scatter_body, bt),
        grid=(n // bt, e, k // bs),
        in_specs=[
            pl.BlockSpec((1, 1, bs), lambda t, ei, si: (ei, 0, si)),
            pl.BlockSpec((1, bs, h), lambda t, ei, si: (ei, si, 0)),
        ],
        out_specs=pl.BlockSpec((bt, h), lambda t, ei, si: (t, 0)),
        out_shape=jax.ShapeDtypeStruct((n, h), jnp.float32),
    )(idx3, yg)


def _scat_body(total, tw, sb_num, yg_hbm, idx_hbm, z_hbm, out_hbm,
               idxv, rowl, tokl, buf, acc, sem):
    cid = lax.axis_index("c")
    sid = lax.axis_index("s")
    wid = sid * 2 + cid
    wbase = wid * (tw * sb_num)
    pltpu.sync_copy(idx_hbm, idxv)
    lanes = lax.iota(jnp.int32, 16)

    def sub_block(sb, _):
        lo = wbase + sb * tw
        pltpu.sync_copy(z_hbm, acc)

        def fill(i, _):
            rowl[pl.ds(i * 16, 16)] = i * 16 + lanes
            return 0
        lax.fori_loop(0, rowl.shape[0] // 16, fill, 0)

        def scan(i, cnt):
            v = idxv[pl.ds(i * 16, 16)]
            m = (v >= lo) & (v < lo + tw)
            pos = plsc.cumsum(m.astype(jnp.int32)) + cnt - 1
            rows = i * 16 + lanes
            plsc.store_scatter(tokl, [pos], v - lo, mask=m)
            plsc.store_scatter(rowl, [pos], rows, mask=m)
            return cnt + plsc.all_reduce_population_count(m)
        cnt = lax.fori_loop(0, total // 16, scan, jnp.zeros((16,), jnp.int32))
        nw = lax.reduce_max(cnt, (0,))

        def g(gi, _):
            rows_v = rowl[pl.ds(gi * 16, 16)]
            pltpu.async_copy(yg_hbm.at[rows_v], buf, sem).wait()
            ta = tokl[pl.ds(gi * 16, 16)]

            def add_row(j, _):
                tok = lax.reduce_max(jnp.where(lanes == j, ta, -1), (0,))
                for c in range(buf.shape[1] // 16):
                    plsc.addupdate(acc.at[tok, pl.ds(c * 16, 16)],
                                   buf[j, pl.ds(c * 16, 16)])
                return 0
            lax.fori_loop(0, jnp.minimum(nw - gi * 16, 16), add_row, 0)
            return 0
        lax.fori_loop(0, (nw + 15) // 16, g, 0)
        pltpu.sync_copy(acc, out_hbm.at[pl.ds(lo, tw)])
        return 0

    lax.fori_loop(0, sb_num, sub_block, 0)


def _scatter_sc(yg_flat, idx_flat, n, tw=32):
    total, h = yg_flat.shape
    zeros = jnp.zeros((tw, h), jnp.float32)
    sb_num = n // (tw * 32)
    lcap = 8 * tw
    mesh = plsc.VectorSubcoreMesh(core_axis_name="c", subcore_axis_name="s")
    f = functools.partial(
        pl.kernel,
        out_type=jax.ShapeDtypeStruct((n, h), jnp.float32),
        mesh=mesh,
        scratch_types=[
            pltpu.VMEM((total,), jnp.int32),
            pltpu.VMEM((lcap,), jnp.int32),
            pltpu.VMEM((lcap,), jnp.int32),
            pltpu.VMEM((16, h), jnp.float32),
            pltpu.VMEM((tw, h), jnp.float32),
            pltpu.SemaphoreType.DMA,
        ],
        compiler_params=pltpu.CompilerParams(needs_layout_passes=False),
    )(functools.partial(_scat_body, total, tw, sb_num))
    return f(yg_flat, idx_flat, zeros)


def _loss_body(total, s_ref, o_ref):
    sums = jnp.sum(s_ref[...], axis=1, keepdims=True)
    loads = sums / total
    e = loads.shape[0]
    m = jnp.sum(loads, axis=0, keepdims=True) / e
    d = loads - m
    o_ref[...] = jnp.sum(d * d, axis=0, keepdims=True) * (e / (e - 1.0))


def _loss(scores, total):
    e, k = scores.shape
    return pl.pallas_call(
        functools.partial(_loss_body, float(total)),
        in_specs=[pl.BlockSpec((e, k), lambda: (0, 0))],
        out_specs=pl.BlockSpec((1, 1), lambda: (0, 0)),
        out_shape=jax.ShapeDtypeStruct((1, 1), jnp.float32),
    )(scores)


def kernel(hidden_states, gate_w, w1, b1, w2, b2):
    b, s, h = hidden_states.shape
    e = gate_w.shape[0]
    n = b * s
    k = int(1.25 * (n // e))

    x = hidden_states.reshape(n, h)
    probs_t = _router(x, gate_w)
    thr, cgt = _thresholds(probs_t, k)
    idx, scores = _compact(probs_t, thr, cgt, k)
    xg = _gather(x, idx.reshape(e * k))
    hh = _mlp_a(xg.reshape(e, k, h), w1, b1.reshape(e, 1, -1))
    yg = _mlp_b(hh, w2, b2.reshape(e, 1, -1), scores.reshape(e, k, 1))
    out = _scatter_sc(yg.reshape(e * k, h), idx.reshape(e * k), n)
    loss = _loss(scores, n)
    return (out.reshape(b, s, h), loss.reshape(()))


# speedup vs baseline: 1.1151x; 1.0007x over previous
import functools
import math

import jax
import jax.numpy as jnp
from jax import lax
from jax.experimental import pallas as pl
from jax.experimental.pallas import tpu as pltpu
from jax.experimental.pallas import tpu_sc as plsc


def _router_body(x_ref, gw_ref, p_ref):
    g = lax.dot_general(
        gw_ref[...], x_ref[...], (((1,), (1,)), ((), ())),
        preferred_element_type=jnp.float32,
    )
    m = jnp.max(g, axis=0, keepdims=True)
    ex = jnp.exp(g - m)
    p_ref[...] = ex / jnp.sum(ex, axis=0, keepdims=True)


def _router(x, gate_w, bm=1024):
    n, h = x.shape
    e = gate_w.shape[0]
    return pl.pallas_call(
        _router_body,
        grid=(n // bm,),
        in_specs=[
            pl.BlockSpec((bm, h), lambda m: (m, 0)),
            pl.BlockSpec((e, h), lambda m: (0, 0)),
        ],
        out_specs=pl.BlockSpec((e, bm), lambda m: (0, m)),
        out_shape=jax.ShapeDtypeStruct((e, n), jnp.float32),
        compiler_params=pltpu.CompilerParams(
            dimension_semantics=("parallel",)),
    )(x, gate_w)


def _threshold_body(k, p_ref, thr_ref, cgt_ref):
    bits = lax.bitcast_convert_type(p_ref[...], jnp.int32)
    e = bits.shape[0]
    t = jnp.zeros((e, 1), jnp.int32)
    for b in range(30, -1, -1):
        cand = t | jnp.int32(1 << b)
        cnt = jnp.sum((bits >= cand).astype(jnp.int32), axis=1, keepdims=True)
        t = jnp.where(cnt >= k, cand, t)
    cgt = jnp.sum((bits > t).astype(jnp.int32), axis=1, keepdims=True)
    thr_ref[...] = jnp.broadcast_to(lax.bitcast_convert_type(t, jnp.float32), (e, 16))
    cgt_ref[...] = jnp.broadcast_to(cgt, (e, 16))


def _thresholds(probs_t, k):
    e, n = probs_t.shape
    return pl.pallas_call(
        functools.partial(_threshold_body, k),
        in_specs=[pl.BlockSpec((e, n), lambda: (0, 0))],
        out_specs=[
            pl.BlockSpec((e, 16), lambda: (0, 0)),
            pl.BlockSpec((e, 16), lambda: (0, 0)),
        ],
        out_shape=[
            jax.ShapeDtypeStruct((e, 16), jnp.float32),
            jax.ShapeDtypeStruct((e, 16), jnp.int32),
        ],
    )(probs_t)


def _compact_body(e_num, n, k, p_hbm, thr_hbm, cgt_hbm, idx_hbm, sc_hbm,
                  pv, iv, sv, tv, cv):
    cid = lax.axis_index("c")
    sid = lax.axis_index("s")
    wid = sid * 2 + cid

    @pl.when(wid < e_num)
    def _():
        pltpu.sync_copy(p_hbm.at[wid], pv)
        pltpu.sync_copy(thr_hbm.at[wid], tv)
        pltpu.sync_copy(cgt_hbm.at[wid], cv)
        t = tv[...]
        cgt = cv[...]
        r = k - cgt

        def body(i, carry):
            cg, ce = carry
            v = pv[pl.ds(i * 16, 16)]
            gt = v > t
            eq = v == t
            pgt = plsc.cumsum(gt.astype(jnp.int32)) + cg
            peq = plsc.cumsum(eq.astype(jnp.int32)) + ce
            sel_eq = eq & (peq <= r)
            slot = jnp.where(gt, pgt - 1, cgt + peq - 1)
            m = gt | sel_eq
            ids = lax.iota(jnp.int32, 16) + i * 16
            plsc.store_scatter(iv, [slot], ids, mask=m)
            plsc.store_scatter(sv, [slot], v, mask=m)
            return (cg + plsc.all_reduce_population_count(gt),
                    ce + plsc.all_reduce_population_count(eq))

        zero16 = jnp.zeros((16,), jnp.int32)
        lax.fori_loop(0, n // 16, body, (zero16, zero16))
        pltpu.sync_copy(iv, idx_hbm.at[wid])
        pltpu.sync_copy(sv, sc_hbm.at[wid])


def _compact(probs_t, thr, cgt, k):
    e, n = probs_t.shape
    mesh = plsc.VectorSubcoreMesh(core_axis_name="c", subcore_axis_name="s")
    f = functools.partial(
        pl.kernel,
        out_type=(jax.ShapeDtypeStruct((e, k), jnp.int32),
                  jax.ShapeDtypeStruct((e, k), jnp.float32)),
        mesh=mesh,
        scratch_types=[
            pltpu.VMEM((n,), jnp.float32),
            pltpu.VMEM((k,), jnp.int32),
            pltpu.VMEM((k,), jnp.float32),
            pltpu.VMEM((16,), jnp.float32),
            pltpu.VMEM((16,), jnp.int32),
        ],
        compiler_params=pltpu.CompilerParams(needs_layout_passes=False),
    )(functools.partial(_compact_body, e, n, k))
    return f(probs_t, thr, cgt)


def _gather_body(rows_w, ch, x_hbm, idx_hbm, xg_hbm, idxv, buf, sem):
    cid = lax.axis_index("c")
    sid = lax.axis_index("s")
    wid = sid * 2 + cid
    base = wid * rows_w
    pltpu.sync_copy(idx_hbm.at[pl.ds(base, rows_w)], idxv)

    def chunk(c, _):
        pltpu.async_copy(x_hbm.at[idxv.at[pl.ds(c * ch, ch)]], buf, sem).wait()
        pltpu.sync_copy(buf, xg_hbm.at[pl.ds(base + c * ch, ch)])
        return 0

    lax.fori_loop(0, rows_w // ch, chunk, 0)


def _gather(x, idx_flat, ch=32):
    n, h = x.shape
    total = idx_flat.shape[0]
    rows_w = total // 32
    mesh = plsc.VectorSubcoreMesh(core_axis_name="c", subcore_axis_name="s")
    f = functools.partial(
        pl.kernel,
        out_type=jax.ShapeDtypeStruct((total, h), jnp.float32),
        mesh=mesh,
        scratch_types=[
            pltpu.VMEM((rows_w,), jnp.int32),
            pltpu.VMEM((ch, h), jnp.float32),
            pltpu.SemaphoreType.DMA,
        ],
        compiler_params=pltpu.CompilerParams(needs_layout_passes=False),
    )(functools.partial(_gather_body, rows_w, ch))
    return f(x, idx_flat)


def _mlp_a_body(xg_ref, w1_ref, b1_ref, h_ref, xb_ref):
    f_idx = pl.program_id(1)

    @pl.when(f_idx == 0)
    def _():
        xb_ref[...] = xg_ref[0].astype(jnp.bfloat16)

    w1b = w1_ref[0].astype(jnp.bfloat16)
    h = lax.dot_general(xb_ref[...], w1b, (((1,), (1,)), ((), ())),
                        preferred_element_type=jnp.float32)
    h = h + b1_ref[0, 0, :][None, :]
    h = 0.5 * h * (1.0 + lax.erf(h * (1.0 / math.sqrt(2.0))))
    h_ref[0] = h.astype(jnp.bfloat16)


def _mlp_a(xg, w1, b1, bf=512):
    e, k, h = xg.shape
    f = w1.shape[1]
    return pl.pallas_call(
        _mlp_a_body,
        grid=(e, f // bf),
        in_specs=[
            pl.BlockSpec((1, k, h), lambda ei, fi: (ei, 0, 0)),
            pl.BlockSpec((1, bf, h), lambda ei, fi: (ei, fi, 0)),
            pl.BlockSpec((1, 1, bf), lambda ei, fi: (ei, 0, fi)),
        ],
        out_specs=pl.BlockSpec((1, k, bf), lambda ei, fi: (ei, 0, fi)),
        out_shape=jax.ShapeDtypeStruct((e, k, f), jnp.bfloat16),
        scratch_shapes=[pltpu.VMEM((k, h), jnp.bfloat16)],
        compiler_params=pltpu.CompilerParams(
            dimension_semantics=("parallel", "arbitrary")),
    )(xg, w1, b1)


def _mlp_b_body(fsteps, h_ref, w2_ref, b2_ref, s_ref, yg_ref):
    f_idx = pl.program_id(2)
    w2b = w2_ref[0].astype(jnp.bfloat16)
    contrib = lax.dot_general(h_ref[0], w2b, (((1,), (1,)), ((), ())),
                              preferred_element_type=jnp.float32)

    @pl.when(f_idx == 0)
    def _():
        yg_ref[0] = contrib

    @pl.when(f_idx > 0)
    def _():
        yg_ref[0] += contrib

    @pl.when(f_idx == fsteps - 1)
    def _():
        yg_ref[0] = (yg_ref[0] + b2_ref[0, 0, :][None, :]) * s_ref[0]


def _mlp_b(hh, w2, b2, scores3, bf=512, nh=2048):
    e, k, f = hh.shape
    h = w2.shape[1]
    fsteps = f // bf
    return pl.pallas_call(
        functools.partial(_mlp_b_body, fsteps),
        grid=(e, h // nh, fsteps),
        in_specs=[
            pl.BlockSpec((1, k, bf), lambda ei, ni, fi: (ei, 0, fi)),
            pl.BlockSpec((1, nh, bf), lambda ei, ni, fi: (ei, ni, fi)),
            pl.BlockSpec((1, 1, nh), lambda ei, ni, fi: (ei, 0, ni)),
            pl.BlockSpec((1, k, 1), lambda ei, ni, fi: (ei, 0, 0)),
        ],
        out_specs=pl.BlockSpec((1, k, nh), lambda ei, ni, fi: (ei, 0, ni)),
        out_shape=jax.ShapeDtypeStruct((e, k, h), jnp.float32),
        compiler_params=pltpu.CompilerParams(
            dimension_semantics=("parallel", "parallel", "arbitrary")),
    )(hh, w2, b2, scores3)


def _scatter_body(bt, idx_ref, yg_ref, out_ref):
    t, e_idx, s_idx = pl.program_id(0), pl.program_id(1), pl.program_id(2)
    bs = idx_ref.shape[2]
    tok = lax.broadcasted_iota(jnp.int32, (bt, bs), 0) + t * bt
    p = (tok == idx_ref[0, 0, :][None, :]).astype(jnp.bfloat16)
    ygb = yg_ref[0].astype(jnp.bfloat16)
    contrib = lax.dot_general(p, ygb, (((1,), (0,)), ((), ())),
                              preferred_element_type=jnp.float32)
    first = (e_idx == 0) & (s_idx == 0)

    @pl.when(first)
    def _():
        out_ref[...] = contrib

    @pl.when(jnp.logical_not(first))
    def _():
        out_ref[...] += contrib


def _scatter(idx3, yg, n, bt=1024, bs=256):
    e, k, h = yg.shape
    return pl.pallas_call(
        functools.partial(_scatter_body, bt),
        grid=(n // bt, e, k // bs),
        in_specs=[
            pl.BlockSpec((1, 1, bs), lambda t, ei, si: (ei, 0, si)),
            pl.BlockSpec((1, bs, h), lambda t, ei, si: (ei, si, 0)),
        ],
        out_specs=pl.BlockSpec((bt, h), lambda t, ei, si: (t, 0)),
        out_shape=jax.ShapeDtypeStruct((n, h), jnp.float32),
    )(idx3, yg)


def _scat_body(total, tw, sb_num, yg_hbm, idx_hbm, z_hbm, out_hbm,
               idxv, rowl, tokl, buf, acc, sem):
    cid = lax.axis_index("c")
    sid = lax.axis_index("s")
    wid = sid * 2 + cid
    wbase = wid * (tw * sb_num)
    pltpu.sync_copy(idx_hbm, idxv)
    lanes = lax.iota(jnp.int32, 16)

    def sub_block(sb, _):
        lo = wbase + sb * tw
        pltpu.sync_copy(z_hbm, acc)

        def fill(i, _):
            rowl[pl.ds(i * 16, 16)] = i * 16 + lanes
            return 0
        lax.fori_loop(0, rowl.shape[0] // 16, fill, 0)

        def scan(i, cnt):
            v = idxv[pl.ds(i * 16, 16)]
            m = (v >= lo) & (v < lo + tw)
            pos = plsc.cumsum(m.astype(jnp.int32)) + cnt - 1
            rows = i * 16 + lanes
            plsc.store_scatter(tokl, [pos], v - lo, mask=m)
            plsc.store_scatter(rowl, [pos], rows, mask=m)
            return cnt + plsc.all_reduce_population_count(m)
        cnt = lax.fori_loop(0, total // 16, scan, jnp.zeros((16,), jnp.int32))
        nw = lax.reduce_max(cnt, (0,))

        def g(gi, _):
            rows_v = rowl[pl.ds(gi * 16, 16)]
            pltpu.async_copy(yg_hbm.at[rows_v], buf, sem).wait()
            ta = tokl[pl.ds(gi * 16, 16)]

            def add_row(j, _):
                tok = lax.reduce_max(jnp.where(lanes == j, ta, -1), (0,))
                for c in range(buf.shape[1] // 16):
                    plsc.addupdate(acc.at[tok, pl.ds(c * 16, 16)],
                                   buf[j, pl.ds(c * 16, 16)])
                return 0
            lax.fori_loop(0, jnp.minimum(nw - gi * 16, 16), add_row, 0)
            return 0
        lax.fori_loop(0, (nw + 15) // 16, g, 0)
        pltpu.sync_copy(acc, out_hbm.at[pl.ds(lo, tw)])
        return 0

    lax.fori_loop(0, sb_num, sub_block, 0)


def _scatter_sc(yg_flat, idx_flat, n, tw=32):
    total, h = yg_flat.shape
    zeros = jnp.zeros((tw, h), jnp.float32)
    sb_num = n // (tw * 32)
    lcap = 8 * tw
    mesh = plsc.VectorSubcoreMesh(core_axis_name="c", subcore_axis_name="s")
    f = functools.partial(
        pl.kernel,
        out_type=jax.ShapeDtypeStruct((n, h), jnp.float32),
        mesh=mesh,
        scratch_types=[
            pltpu.VMEM((total,), jnp.int32),
            pltpu.VMEM((lcap,), jnp.int32),
            pltpu.VMEM((lcap,), jnp.int32),
            pltpu.VMEM((16, h), jnp.float32),
            pltpu.VMEM((tw, h), jnp.float32),
            pltpu.SemaphoreType.DMA,
        ],
        compiler_params=pltpu.CompilerParams(needs_layout_passes=False),
    )(functools.partial(_scat_body, total, tw, sb_num))
    return f(yg_flat, idx_flat, zeros)


def _loss_body(total, s_ref, o_ref):
    sums = jnp.sum(s_ref[...], axis=1, keepdims=True)
    loads = sums / total
    e = loads.shape[0]
    m = jnp.sum(loads, axis=0, keepdims=True) / e
    d = loads - m
    o_ref[...] = jnp.sum(d * d, axis=0, keepdims=True) * (e / (e - 1.0))


def _loss(scores, total):
    e, k = scores.shape
    return pl.pallas_call(
        functools.partial(_loss_body, float(total)),
        in_specs=[pl.BlockSpec((e, k), lambda: (0, 0))],
        out_specs=pl.BlockSpec((1, 1), lambda: (0, 0)),
        out_shape=jax.ShapeDtypeStruct((1, 1), jnp.float32),
    )(scores)


def kernel(hidden_states, gate_w, w1, b1, w2, b2):
    b, s, h = hidden_states.shape
    e = gate_w.shape[0]
    n = b * s
    k = int(1.25 * (n // e))

    x = hidden_states.reshape(n, h)
    probs_t = _router(x, gate_w)
    thr, cgt = _thresholds(probs_t, k)
    idx, scores = _compact(probs_t, thr, cgt, k)
    xg = _gather(x, idx.reshape(e * k))
    hh = _mlp_a(xg.reshape(e, k, h), w1, b1.reshape(e, 1, -1))
    yg = _mlp_b(hh, w2, b2.reshape(e, 1, -1), scores.reshape(e, k, 1))
    out = _scatter_sc(yg.reshape(e * k, h), idx.reshape(e * k), n)
    loss = _loss(scores, n)
    return (out.reshape(b, s, h), loss.reshape(()))


# speedup vs baseline: 1.1584x; 1.0388x over previous
import functools
import math

import jax
import jax.numpy as jnp
from jax import lax
from jax.experimental import pallas as pl
from jax.experimental.pallas import tpu as pltpu
from jax.experimental.pallas import tpu_sc as plsc


def _router_body(x_ref, gw_ref, p_ref):
    g = lax.dot_general(
        gw_ref[...], x_ref[...], (((1,), (1,)), ((), ())),
        preferred_element_type=jnp.float32,
    )
    m = jnp.max(g, axis=0, keepdims=True)
    ex = jnp.exp(g - m)
    p_ref[...] = ex / jnp.sum(ex, axis=0, keepdims=True)


def _router(x, gate_w, bm=1024):
    n, h = x.shape
    e = gate_w.shape[0]
    return pl.pallas_call(
        _router_body,
        grid=(n // bm,),
        in_specs=[
            pl.BlockSpec((bm, h), lambda m: (m, 0)),
            pl.BlockSpec((e, h), lambda m: (0, 0)),
        ],
        out_specs=pl.BlockSpec((e, bm), lambda m: (0, m)),
        out_shape=jax.ShapeDtypeStruct((e, n), jnp.float32),
        compiler_params=pltpu.CompilerParams(
            dimension_semantics=("parallel",)),
    )(x, gate_w)


def _threshold_body(k, p_ref, thr_ref, cgt_ref):
    bits = lax.bitcast_convert_type(p_ref[...], jnp.int32)
    e = bits.shape[0]
    t = jnp.zeros((e, 1), jnp.int32)
    for b in range(30, -1, -1):
        cand = t | jnp.int32(1 << b)
        cnt = jnp.sum((bits >= cand).astype(jnp.int32), axis=1, keepdims=True)
        t = jnp.where(cnt >= k, cand, t)
    cgt = jnp.sum((bits > t).astype(jnp.int32), axis=1, keepdims=True)
    thr_ref[...] = jnp.broadcast_to(lax.bitcast_convert_type(t, jnp.float32), (e, 16))
    cgt_ref[...] = jnp.broadcast_to(cgt, (e, 16))


def _thresholds(probs_t, k):
    e, n = probs_t.shape
    return pl.pallas_call(
        functools.partial(_threshold_body, k),
        in_specs=[pl.BlockSpec((e, n), lambda: (0, 0))],
        out_specs=[
            pl.BlockSpec((e, 16), lambda: (0, 0)),
            pl.BlockSpec((e, 16), lambda: (0, 0)),
        ],
        out_shape=[
            jax.ShapeDtypeStruct((e, 16), jnp.float32),
            jax.ShapeDtypeStruct((e, 16), jnp.int32),
        ],
    )(probs_t)


def _compact_body(e_num, n, k, p_hbm, thr_hbm, cgt_hbm, idx_hbm, sc_hbm,
                  pv, iv, sv, tv, cv):
    cid = lax.axis_index("c")
    sid = lax.axis_index("s")
    wid = sid * 2 + cid

    @pl.when(wid < e_num)
    def _():
        pltpu.sync_copy(p_hbm.at[wid], pv)
        pltpu.sync_copy(thr_hbm.at[wid], tv)
        pltpu.sync_copy(cgt_hbm.at[wid], cv)
        t = tv[...]
        cgt = cv[...]
        r = k - cgt

        def body(i, carry):
            cg, ce = carry
            v = pv[pl.ds(i * 16, 16)]
            gt = v > t
            eq = v == t
            pgt = plsc.cumsum(gt.astype(jnp.int32)) + cg
            peq = plsc.cumsum(eq.astype(jnp.int32)) + ce
            sel_eq = eq & (peq <= r)
            slot = jnp.where(gt, pgt - 1, cgt + peq - 1)
            m = gt | sel_eq
            ids = lax.iota(jnp.int32, 16) + i * 16
            plsc.store_scatter(iv, [slot], ids, mask=m)
            plsc.store_scatter(sv, [slot], v, mask=m)
            return (cg + plsc.all_reduce_population_count(gt),
                    ce + plsc.all_reduce_population_count(eq))

        zero16 = jnp.zeros((16,), jnp.int32)
        lax.fori_loop(0, n // 16, body, (zero16, zero16))
        pltpu.sync_copy(iv, idx_hbm.at[wid])
        pltpu.sync_copy(sv, sc_hbm.at[wid])


def _compact(probs_t, thr, cgt, k):
    e, n = probs_t.shape
    mesh = plsc.VectorSubcoreMesh(core_axis_name="c", subcore_axis_name="s")
    f = functools.partial(
        pl.kernel,
        out_type=(jax.ShapeDtypeStruct((e, k), jnp.int32),
                  jax.ShapeDtypeStruct((e, k), jnp.float32)),
        mesh=mesh,
        scratch_types=[
            pltpu.VMEM((n,), jnp.float32),
            pltpu.VMEM((k,), jnp.int32),
            pltpu.VMEM((k,), jnp.float32),
            pltpu.VMEM((16,), jnp.float32),
            pltpu.VMEM((16,), jnp.int32),
        ],
        compiler_params=pltpu.CompilerParams(needs_layout_passes=False),
    )(functools.partial(_compact_body, e, n, k))
    return f(probs_t, thr, cgt)


def _gather_body(rows_w, ch, x_hbm, idx_hbm, xg_hbm, idxv, buf0, buf1, sem0, sem1):
    cid = lax.axis_index("c")
    sid = lax.axis_index("s")
    wid = sid * 2 + cid
    base = wid * rows_w
    pltpu.sync_copy(idx_hbm.at[pl.ds(base, rows_w)], idxv)
    nch = rows_w // ch
    cp0 = pltpu.make_async_copy(x_hbm.at[idxv.at[pl.ds(0, ch)]], buf0, sem0)
    cp0.start()

    def pair(p, _):
        c0 = p * 2
        @pl.when(c0 + 1 < nch)
        def _():
            pltpu.make_async_copy(
                x_hbm.at[idxv.at[pl.ds((c0 + 1) * ch, ch)]], buf1, sem1
            ).start()
        pltpu.make_async_copy(
            x_hbm.at[idxv.at[pl.ds(c0 * ch, ch)]], buf0, sem0).wait()
        pltpu.sync_copy(buf0, xg_hbm.at[pl.ds(base + c0 * ch, ch)])

        @pl.when(c0 + 1 < nch)
        def _():
            @pl.when(c0 + 2 < nch)
            def _():
                pltpu.make_async_copy(
                    x_hbm.at[idxv.at[pl.ds((c0 + 2) * ch, ch)]], buf0, sem0
                ).start()
            pltpu.make_async_copy(
                x_hbm.at[idxv.at[pl.ds((c0 + 1) * ch, ch)]], buf1, sem1).wait()
            pltpu.sync_copy(buf1, xg_hbm.at[pl.ds(base + (c0 + 1) * ch, ch)])
        return 0

    lax.fori_loop(0, (nch + 1) // 2, pair, 0)


def _gather(x, idx_flat, ch=16):
    n, h = x.shape
    total = idx_flat.shape[0]
    rows_w = total // 32
    mesh = plsc.VectorSubcoreMesh(core_axis_name="c", subcore_axis_name="s")
    f = functools.partial(
        pl.kernel,
        out_type=jax.ShapeDtypeStruct((total, h), jnp.float32),
        mesh=mesh,
        scratch_types=[
            pltpu.VMEM((rows_w,), jnp.int32),
            pltpu.VMEM((ch, h), jnp.float32),
            pltpu.VMEM((ch, h), jnp.float32),
            pltpu.SemaphoreType.DMA,
            pltpu.SemaphoreType.DMA,
        ],
        compiler_params=pltpu.CompilerParams(needs_layout_passes=False),
    )(functools.partial(_gather_body, rows_w, ch))
    return f(x, idx_flat)


def _mlp_a_body(xg_ref, w1_ref, b1_ref, h_ref, xb_ref):
    f_idx = pl.program_id(1)

    @pl.when(f_idx == 0)
    def _():
        xb_ref[...] = xg_ref[0].astype(jnp.bfloat16)

    w1b = w1_ref[0].astype(jnp.bfloat16)
    h = lax.dot_general(xb_ref[...], w1b, (((1,), (1,)), ((), ())),
                        preferred_element_type=jnp.float32)
    h = h + b1_ref[0, 0, :][None, :]
    h = 0.5 * h * (1.0 + lax.erf(h * (1.0 / math.sqrt(2.0))))
    h_ref[0] = h.astype(jnp.bfloat16)


def _mlp_a(xg, w1, b1, bf=512):
    e, k, h = xg.shape
    f = w1.shape[1]
    return pl.pallas_call(
        _mlp_a_body,
        grid=(e, f // bf),
        in_specs=[
            pl.BlockSpec((1, k, h), lambda ei, fi: (ei, 0, 0)),
            pl.BlockSpec((1, bf, h), lambda ei, fi: (ei, fi, 0)),
            pl.BlockSpec((1, 1, bf), lambda ei, fi: (ei, 0, fi)),
        ],
        out_specs=pl.BlockSpec((1, k, bf), lambda ei, fi: (ei, 0, fi)),
        out_shape=jax.ShapeDtypeStruct((e, k, f), jnp.bfloat16),
        scratch_shapes=[pltpu.VMEM((k, h), jnp.bfloat16)],
        compiler_params=pltpu.CompilerParams(
            dimension_semantics=("parallel", "arbitrary")),
    )(xg, w1, b1)


def _mlp_b_body(fsteps, h_ref, w2_ref, b2_ref, s_ref, yg_ref):
    f_idx = pl.program_id(2)
    w2b = w2_ref[0].astype(jnp.bfloat16)
    contrib = lax.dot_general(h_ref[0], w2b, (((1,), (1,)), ((), ())),
                              preferred_element_type=jnp.float32)

    @pl.when(f_idx == 0)
    def _():
        yg_ref[0] = contrib

    @pl.when(f_idx > 0)
    def _():
        yg_ref[0] += contrib

    @pl.when(f_idx == fsteps - 1)
    def _():
        yg_ref[0] = (yg_ref[0] + b2_ref[0, 0, :][None, :]) * s_ref[0]


def _mlp_b(hh, w2, b2, scores3, bf=512, nh=2048):
    e, k, f = hh.shape
    h = w2.shape[1]
    fsteps = f // bf
    return pl.pallas_call(
        functools.partial(_mlp_b_body, fsteps),
        grid=(e, h // nh, fsteps),
        in_specs=[
            pl.BlockSpec((1, k, bf), lambda ei, ni, fi: (ei, 0, fi)),
            pl.BlockSpec((1, nh, bf), lambda ei, ni, fi: (ei, ni, fi)),
            pl.BlockSpec((1, 1, nh), lambda ei, ni, fi: (ei, 0, ni)),
            pl.BlockSpec((1, k, 1), lambda ei, ni, fi: (ei, 0, 0)),
        ],
        out_specs=pl.BlockSpec((1, k, nh), lambda ei, ni, fi: (ei, 0, ni)),
        out_shape=jax.ShapeDtypeStruct((e, k, h), jnp.float32),
        compiler_params=pltpu.CompilerParams(
            dimension_semantics=("parallel", "parallel", "arbitrary")),
    )(hh, w2, b2, scores3)


def _scatter_body(bt, idx_ref, yg_ref, out_ref):
    t, e_idx, s_idx = pl.program_id(0), pl.program_id(1), pl.program_id(2)
    bs = idx_ref.shape[2]
    tok = lax.broadcasted_iota(jnp.int32, (bt, bs), 0) + t * bt
    p = (tok == idx_ref[0, 0, :][None, :]).astype(jnp.bfloat16)
    ygb = yg_ref[0].astype(jnp.bfloat16)
    contrib = lax.dot_general(p, ygb, (((1,), (0,)), ((), ())),
                              preferred_element_type=jnp.float32)
    first = (e_idx == 0) & (s_idx == 0)

    @pl.when(first)
    def _():
        out_ref[...] = contrib

    @pl.when(jnp.logical_not(first))
    def _():
        out_ref[...] += contrib


def _scatter(idx3, yg, n, bt=1024, bs=256):
    e, k, h = yg.shape
    return pl.pallas_call(
        functools.partial(_scatter_body, bt),
        grid=(n // bt, e, k // bs),
        in_specs=[
            pl.BlockSpec((1, 1, bs), lambda t, ei, si: (ei, 0, si)),
            pl.BlockSpec((1, bs, h), lambda t, ei, si: (ei, si, 0)),
        ],
        out_specs=pl.BlockSpec((bt, h), lambda t, ei, si: (t, 0)),
        out_shape=jax.ShapeDtypeStruct((n, h), jnp.float32),
    )(idx3, yg)


def _scat_body(total, tw, sb_num, yg_hbm, idx_hbm, z_hbm, out_hbm,
               idxv, lwr, lwt, rowl, tokl, buf, acc, sem):
    cid = lax.axis_index("c")
    sid = lax.axis_index("s")
    wid = sid * 2 + cid
    rng = tw * sb_num
    wbase = wid * rng
    pltpu.sync_copy(idx_hbm, idxv)
    lanes = lax.iota(jnp.int32, 16)

    def fill1(i, _):
        lwt[pl.ds(i * 16, 16)] = jnp.full((16,), rng, jnp.int32)
        return 0
    lax.fori_loop(0, lwt.shape[0] // 16, fill1, 0)

    def scan1(i, cnt):
        v = idxv[pl.ds(i * 16, 16)]
        m = (v >= wbase) & (v < wbase + rng)
        pos = plsc.cumsum(m.astype(jnp.int32)) + cnt - 1
        plsc.store_scatter(lwt, [pos], v - wbase, mask=m)
        plsc.store_scatter(lwr, [pos], i * 16 + lanes, mask=m)
        return cnt + plsc.all_reduce_population_count(m)
    c1 = lax.fori_loop(0, total // 16, scan1, jnp.zeros((16,), jnp.int32))
    nv1 = (lax.reduce_max(c1, (0,)) + 15) // 16

    def sub_block(sb, _):
        lo = sb * tw
        pltpu.sync_copy(z_hbm, acc)

        def fill(i, _):
            rowl[pl.ds(i * 16, 16)] = i * 16 + lanes
            return 0
        lax.fori_loop(0, rowl.shape[0] // 16, fill, 0)

        def scan(i, cnt):
            v = lwt[pl.ds(i * 16, 16)]
            m = (v >= lo) & (v < lo + tw)
            r = lwr[pl.ds(i * 16, 16)]
            pos = plsc.cumsum(m.astype(jnp.int32)) + cnt - 1
            plsc.store_scatter(tokl, [pos], v - lo, mask=m)
            plsc.store_scatter(rowl, [pos], r, mask=m)
            return cnt + plsc.all_reduce_population_count(m)
        cnt = lax.fori_loop(0, nv1, scan, jnp.zeros((16,), jnp.int32))
        nw = lax.reduce_max(cnt, (0,))

        def g(gi, _):
            rows_v = rowl[pl.ds(gi * 16, 16)]
            pltpu.async_copy(yg_hbm.at[rows_v], buf, sem).wait()
            ta = tokl[pl.ds(gi * 16, 16)]

            def add_row(j, _):
                tok = lax.reduce_max(jnp.where(lanes == j, ta, -1), (0,))
                for c in range(buf.shape[1] // 16):
                    plsc.addupdate(acc.at[tok, pl.ds(c * 16, 16)],
                                   buf[j, pl.ds(c * 16, 16)])
                return 0
            lax.fori_loop(0, jnp.minimum(nw - gi * 16, 16), add_row, 0)
            return 0
        lax.fori_loop(0, (nw + 15) // 16, g, 0)
        pltpu.sync_copy(acc, out_hbm.at[pl.ds(wbase + lo, tw)])
        return 0

    lax.fori_loop(0, sb_num, sub_block, 0)


def _scatter_sc(yg_flat, idx_flat, n, tw=32):
    total, h = yg_flat.shape
    zeros = jnp.zeros((tw, h), jnp.float32)
    sb_num = n // (tw * 32)
    lcap = 8 * tw
    mesh = plsc.VectorSubcoreMesh(core_axis_name="c", subcore_axis_name="s")
    f = functools.partial(
        pl.kernel,
        out_type=jax.ShapeDtypeStruct((n, h), jnp.float32),
        mesh=mesh,
        scratch_types=[
            pltpu.VMEM((total,), jnp.int32),
            pltpu.VMEM((8 * tw * 8,), jnp.int32),
            pltpu.VMEM((8 * tw * 8,), jnp.int32),
            pltpu.VMEM((lcap,), jnp.int32),
            pltpu.VMEM((lcap,), jnp.int32),
            pltpu.VMEM((16, h), jnp.float32),
            pltpu.VMEM((tw, h), jnp.float32),
            pltpu.SemaphoreType.DMA,
        ],
        compiler_params=pltpu.CompilerParams(needs_layout_passes=False),
    )(functools.partial(_scat_body, total, tw, sb_num))
    return f(yg_flat, idx_flat, zeros)


def _loss_body(total, s_ref, o_ref):
    sums = jnp.sum(s_ref[...], axis=1, keepdims=True)
    loads = sums / total
    e = loads.shape[0]
    m = jnp.sum(loads, axis=0, keepdims=True) / e
    d = loads - m
    o_ref[...] = jnp.sum(d * d, axis=0, keepdims=True) * (e / (e - 1.0))


def _loss(scores, total):
    e, k = scores.shape
    return pl.pallas_call(
        functools.partial(_loss_body, float(total)),
        in_specs=[pl.BlockSpec((e, k), lambda: (0, 0))],
        out_specs=pl.BlockSpec((1, 1), lambda: (0, 0)),
        out_shape=jax.ShapeDtypeStruct((1, 1), jnp.float32),
    )(scores)


def kernel(hidden_states, gate_w, w1, b1, w2, b2):
    b, s, h = hidden_states.shape
    e = gate_w.shape[0]
    n = b * s
    k = int(1.25 * (n // e))

    x = hidden_states.reshape(n, h)
    probs_t = _router(x, gate_w)
    thr, cgt = _thresholds(probs_t, k)
    idx, scores = _compact(probs_t, thr, cgt, k)
    xg = _gather(x, idx.reshape(e * k))
    hh = _mlp_a(xg.reshape(e, k, h), w1, b1.reshape(e, 1, -1))
    yg = _mlp_b(hh, w2, b2.reshape(e, 1, -1), scores.reshape(e, k, 1))
    out = _scatter_sc(yg.reshape(e * k, h), idx.reshape(e * k), n)
    loss = _loss(scores, n)
    return (out.reshape(b, s, h), loss.reshape(()))
